# Initial kernel scaffold; baseline (speedup 1.0000x reference)
import jax
import jax.numpy as jnp
from jax.experimental import pallas as pl


def kernel(x, edge_index, edge_label, node_emb, Wl1, bl1, Wr1, Wl2, bl2, Wr2, W1, b1, W2, b2):
    raise NotImplementedError("write your pallas kernel here")


# speedup vs baseline: 14.4716x; 14.4716x over previous
import functools
import jax
import jax.numpy as jnp
from jax import lax
from jax.experimental import pallas as pl
from jax.experimental.pallas import tpu as pltpu
from jax.experimental.pallas import tpu_sc as plsc

N = 50000
E = 800000
D = 64
B = 4096
H = 2 * B
NPAD = 50176
BLK = 512
NC = 2
NS = 16
NW = NC * NS
EC = 25088
EPAD = NW * EC
HR = 98
FCAP = EC + 128
CNT_SZ = 4 * NPAD
CSLC = CNT_SZ // NS
ACC_R = 8448
ZR = ACC_R // NS

def _vgather(x, idx):
    return lax.gather(
        x, idx[:, None],
        lax.GatherDimensionNumbers(
            offset_dims=(), collapsed_slice_dims=(0,), start_index_map=(0,)),
        slice_sizes=(1,),
        mode=lax.GatherScatterMode.PROMISE_IN_BOUNDS)


_MESH = plsc.VectorSubcoreMesh(
    core_axis_name="c", subcore_axis_name="s", num_cores=NC, num_subcores=NS)


def _stage_a_body(srcp, dstp, xflat, cnt_out, fpk_out, nb_out,
                  sdh, ddh, cls, idx2, tgt, pk, zf, nbst, cnt_acc, fp_sh, sem):
    c = lax.axis_index("c")
    s = lax.axis_index("s")
    w = c * NS + s
    fbase = s * FCAP

    def _zf_zero(i, _):
        zf[pl.ds(i * 16, 16)] = jnp.zeros((16,), jnp.float32)
        return 0
    lax.fori_loop(0, CSLC // 16, _zf_zero, 0)
    pltpu.sync_copy(zf, cnt_acc.at[pl.ds(s * CSLC, CSLC)])

    def _ones(i, _):
        zf[pl.ds(i * 16, 16)] = jnp.ones((16,), jnp.float32)
        return 0
    lax.fori_loop(0, 8, _ones, 0)
    plsc.subcore_barrier()

    def _half(h, cur):
        pltpu.sync_copy(srcp.at[w, h], sdh)
        pltpu.sync_copy(dstp.at[w, h], ddh)

        def _gs(r, _):
            pltpu.make_async_copy(xflat.at[sdh.at[r]], cls.at[r], sem).start()
            return 0
        lax.fori_loop(0, HR, _gs, 0)

        def _gw(r, _):
            pltpu.make_async_copy(xflat.at[sdh.at[0]], cls.at[0], sem).wait()
            return 0
        lax.fori_loop(0, HR, _gw, 0)

        def _idx(i, _):
            r = i >> 3
            col = (i & 7) * 16
            d16 = ddh[r, pl.ds(col, 16)]
            c16 = cls[r, pl.ds(col, 16)]
            idx2[r, pl.ds(col, 16)] = d16 * 4 + c16
            return 0
        lax.fori_loop(0, HR * 8, _idx, 0)

        def _ss(r, _):
            pltpu.make_async_copy(
                zf.at[pl.ds(0, 128)], cnt_acc.at[idx2.at[r]], sem
            ).start(add=True)
            return 0
        lax.fori_loop(0, HR, _ss, 0)

        def _sw(r, _):
            pltpu.make_async_copy(
                zf.at[pl.ds(0, 128)], cnt_acc.at[idx2.at[0]], sem).wait()
            return 0
        lax.fori_loop(0, HR, _sw, 0)

        def _flt(i, cur):
            r = i >> 3
            col = (i & 7) * 16
            s16 = sdh[r, pl.ds(col, 16)]
            d16 = ddh[r, pl.ds(col, 16)]
            m = d16 < H
            packed = s16 * (2 * H) + jnp.where(m, d16, 0)
            lane = lax.iota(jnp.int32, 16)
            pos = jnp.where(m, 1, 0).astype(jnp.int32)
            for k in (1, 2, 4, 8):
                sh = _vgather(pos, jnp.maximum(lane - k, 0))
                pos = pos + jnp.where(lane >= k, sh, 0)
            tgt[r, pl.ds(col, 16)] = fbase + jnp.where(
                m, cur + pos - 1, FCAP - 1)
            pk[r, pl.ds(col, 16)] = packed
            pcv = _vgather(pos, jnp.full((16,), 15, jnp.int32))
            return cur + pcv
        cur = lax.fori_loop(0, HR * 8, _flt, cur)

        def _fsc(r, _):
            pltpu.make_async_copy(
                pk.at[r], fp_sh.at[tgt.at[r]], sem).start()
            return 0
        lax.fori_loop(0, HR, _fsc, 0)

        def _fsw(r, _):
            pltpu.make_async_copy(pk.at[0], fp_sh.at[tgt.at[0]], sem).wait()
            return 0
        lax.fori_loop(0, HR, _fsw, 0)
        return cur

    lane16 = lax.iota(jnp.int32, 16)
    cur = _half(0, jnp.zeros((16,), jnp.int32))
    cur = _half(1, cur)

    for k in range(8):
        tgt[0, pl.ds(k * 16, 16)] = fbase + cur + lane16 + k * 16
        pk[0, pl.ds(k * 16, 16)] = jnp.full((16,), H, jnp.int32)
    pltpu.sync_copy(pk.at[0], fp_sh.at[tgt.at[0]])

    nb128 = lax.shift_right_logical(cur + 127, 7)

    for k in range(8):
        nbst[pl.ds(k * 16, 16)] = nb128
    pltpu.sync_copy(nbst, nb_out.at[w])
    pltpu.sync_copy(fp_sh.at[pl.ds(fbase, FCAP)], fpk_out.at[w])

    plsc.subcore_barrier()
    pltpu.sync_copy(cnt_acc.at[pl.ds(s * CSLC, CSLC)],
                    cnt_out.at[c, pl.ds(s * CSLC, CSLC)])


_stage_a = functools.partial(
    pl.kernel,
    out_type=(
        jax.ShapeDtypeStruct((NC, CNT_SZ), jnp.float32),
        jax.ShapeDtypeStruct((NW, FCAP), jnp.int32),
        jax.ShapeDtypeStruct((NW, 128), jnp.int32),
    ),
    mesh=_MESH,
    scratch_types=(
        pltpu.VMEM((HR, 128), jnp.int32),
        pltpu.VMEM((HR, 128), jnp.int32),
        pltpu.VMEM((HR, 128), jnp.int32),
        pltpu.VMEM((HR, 128), jnp.int32),
        pltpu.VMEM((HR, 128), jnp.int32),
        pltpu.VMEM((HR, 128), jnp.int32),
        pltpu.VMEM((CSLC,), jnp.float32),
        pltpu.VMEM((128,), jnp.int32),
        pltpu.VMEM_SHARED((CNT_SZ,), jnp.float32),
        pltpu.VMEM_SHARED((NS * FCAP,), jnp.int32),
        pltpu.SemaphoreType.DMA,
    ),
)(_stage_a_body)


def _stage_c_body(z1, fpk, nb, s2_out, fp_v, fsr, fdr, rows, nbv, acc, sem):
    c = lax.axis_index("c")
    s = lax.axis_index("s")
    w = c * NS + s

    def _rz_all(i, _):
        r = i >> 3
        col = (i & 7) * 16
        rows[r, pl.ds(col, 16)] = jnp.zeros((16,), jnp.float32)
        return 0
    lax.fori_loop(0, 128 * 8, _rz_all, 0)
    base = s * ZR
    for k in range(4):
        pltpu.sync_copy(rows, acc.at[pl.ds(base + k * 128, 128)])
    pltpu.sync_copy(rows.at[pl.ds(0, 16)], acc.at[pl.ds(base + 512, 16)])
    plsc.subcore_barrier()

    pltpu.sync_copy(fpk.at[w], fp_v)
    pltpu.sync_copy(nb.at[w, pl.ds(0, 16)], nbv)
    n128 = nbv[...][0]

    def _batch(b, _):
        for k in range(8):
            v = fp_v[pl.ds(b * 128 + k * 16, 16)]
            fsr[pl.ds(k * 16, 16)] = lax.shift_right_logical(v, 14)
            fdr[pl.ds(k * 16, 16)] = v & (2 * H - 1)
        g = pltpu.make_async_copy(z1.at[fsr], rows, sem)
        g.start()
        g.wait()
        for k in range(8):
            dvec = fdr[pl.ds(k * 16, 16)]
            pltpu.sync_copy(rows.at[pl.ds(k * 16, 16)], acc.at[dvec], add=True)
        return 0
    lax.fori_loop(0, n128, _batch, 0)

    plsc.subcore_barrier()
    pltpu.sync_copy(acc.at[pl.ds(s * 512, 512)],
                    s2_out.at[c, pl.ds(s * 512, 512)])


_stage_c = functools.partial(
    pl.kernel,
    out_type=jax.ShapeDtypeStruct((NC, H, 128), jnp.float32),
    mesh=_MESH,
    scratch_types=(
        pltpu.VMEM((FCAP,), jnp.int32),
        pltpu.VMEM((128,), jnp.int32),
        pltpu.VMEM((128,), jnp.int32),
        pltpu.VMEM((128, 128), jnp.float32),
        pltpu.VMEM((16,), jnp.int32),
        pltpu.VMEM_SHARED((ACC_R, 128), jnp.float32),
        pltpu.SemaphoreType.DMA,
    ),
)(_stage_c_body)


def _stage_b_body(cnt_ref, x_ref, m1_ref, r1_ref, bl1_ref, out_ref):
    cnt = cnt_ref[0] + cnt_ref[1]
    deg = jnp.sum(cnt, axis=1, keepdims=True)
    invd = 1.0 / jnp.maximum(deg, 1.0)
    a = cnt * invd
    xv = x_ref[...]
    out = jnp.broadcast_to(bl1_ref[...], (BLK, D))
    for c in range(4):
        out = out + a[:, c:c + 1] * m1_ref[c:c + 1, :]
        out = out + jnp.where(xv == c, 1.0, 0.0) * r1_ref[c:c + 1, :]
    norm = jnp.sqrt(jnp.sum(out * out, axis=1, keepdims=True))
    out = out / jnp.maximum(norm, 1e-12)
    out_ref[...] = jnp.concatenate(
        [jnp.maximum(out, 0.0), jnp.zeros((BLK, 128 - D), jnp.float32)],
        axis=1)


def _stage_b(cnt2, x_pad, m1, r1, bl1):
    grid = (NPAD // BLK,)
    return pl.pallas_call(
        _stage_b_body,
        grid=grid,
        in_specs=[
            pl.BlockSpec((2, BLK, 4), lambda i: (0, i, 0)),
            pl.BlockSpec((BLK, 1), lambda i: (i, 0)),
            pl.BlockSpec((4, D), lambda i: (0, 0)),
            pl.BlockSpec((4, D), lambda i: (0, 0)),
            pl.BlockSpec((1, D), lambda i: (0, 0)),
        ],
        out_specs=pl.BlockSpec((BLK, 128), lambda i: (i, 0)),
        out_shape=jax.ShapeDtypeStruct((NPAD, 128), jnp.float32),
    )(cnt2, x_pad, m1, r1, bl1)


def _stage_d_body(s2_ref, cnt_ref, z1_ref, wl2_ref, wr2_ref, bl2_ref,
                  w1_ref, b1_ref, w2_ref, b2_ref, out_ref):
    s = s2_ref[0, :, :D] + s2_ref[1, :, :D]
    cnt = cnt_ref[0] + cnt_ref[1]
    deg = jnp.sum(cnt, axis=1, keepdims=True)
    aggr = s * (1.0 / jnp.maximum(deg, 1.0))
    z1s = z1_ref[:, :D]
    dn = (((1,), (1,)), ((), ()))
    out = (lax.dot_general(aggr, wl2_ref[...], dn,
                           preferred_element_type=jnp.float32)
           + lax.dot_general(z1s, wr2_ref[...], dn,
                             preferred_element_type=jnp.float32)
           + bl2_ref[...])
    norm = jnp.sqrt(jnp.sum(out * out, axis=1, keepdims=True))
    z2 = jnp.maximum(out / jnp.maximum(norm, 1e-12), 0.0)
    za = z2[:B]
    zb = z2[B:]
    w1a = w1_ref[:, :D]
    w1b = w1_ref[:, D:]
    h = (lax.dot_general(za, w1a, dn, preferred_element_type=jnp.float32)
         + lax.dot_general(zb, w1b, dn, preferred_element_type=jnp.float32)
         + b1_ref[...])
    h = jnp.maximum(h, 0.0)
    pred = jnp.sum(h * w2_ref[...], axis=1, keepdims=True) + b2_ref[...]
    out_ref[...] = pred


def _stage_d(s2, cnt2h, z1h, wl2, wr2, bl2, w1, b1, w2, b2):
    return pl.pallas_call(
        _stage_d_body,
        in_specs=[
            pl.BlockSpec((2, H, 128), lambda: (0, 0, 0)),
            pl.BlockSpec((2, H, 4), lambda: (0, 0, 0)),
            pl.BlockSpec((H, 128), lambda: (0, 0)),
            pl.BlockSpec((D, D), lambda: (0, 0)),
            pl.BlockSpec((D, D), lambda: (0, 0)),
            pl.BlockSpec((1, D), lambda: (0, 0)),
            pl.BlockSpec((D, 2 * D), lambda: (0, 0)),
            pl.BlockSpec((1, D), lambda: (0, 0)),
            pl.BlockSpec((1, D), lambda: (0, 0)),
            pl.BlockSpec((1, 1), lambda: (0, 0)),
        ],
        out_specs=pl.BlockSpec((B, 1), lambda: (0, 0)),
        out_shape=jax.ShapeDtypeStruct((B, 1), jnp.float32),
    )(s2, cnt2h, z1h, wl2, wr2, bl2, w1, b1, w2, b2)


def kernel(x, edge_index, edge_label, node_emb,
           Wl1, bl1, Wr1, Wl2, bl2, Wr2, W1, b1, W2, b2):
    m1 = node_emb @ Wl1.T
    r1 = node_emb @ Wr1.T
    x_pad = jnp.pad(x, ((0, NPAD - N), (0, 0)))
    xflat = x[:, 0]

    padn = EPAD - E
    src_p = jnp.concatenate(
        [edge_index[0], jnp.zeros((padn,), jnp.int32)]).reshape(NW, 2, HR, 128)
    dst_p = jnp.concatenate(
        [edge_index[1],
         jnp.full((padn,), NPAD - 1, jnp.int32)]).reshape(NW, 2, HR, 128)

    cnt_flat, fpk, nb = _stage_a(src_p, dst_p, xflat)
    cnt2 = cnt_flat.reshape(NC, NPAD, 4)

    z1 = _stage_b(cnt2, x_pad, m1, r1, bl1.reshape(1, D))
    s2 = _stage_c(z1, fpk, nb)
    pred = _stage_d(s2, cnt2[:, :H, :], z1[:H],
                    Wl2, Wr2, bl2.reshape(1, D),
                    W1, b1.reshape(1, D), W2.reshape(1, D),
                    b2.reshape(1, 1))
    return (pred, edge_label)


# speedup vs baseline: 14.8006x; 1.0227x over previous
import functools
import jax
import jax.numpy as jnp
from jax import lax
from jax.experimental import pallas as pl
from jax.experimental.pallas import tpu as pltpu
from jax.experimental.pallas import tpu_sc as plsc

N = 50000
E = 800000
D = 64
B = 4096
H = 2 * B
NPAD = 50176
BLK = 512
NC = 2
NS = 16
NW = NC * NS
EC = 25088
EPAD = NW * EC
HR = 98
FCAP = EC + 128
CNT_SZ = 4 * NPAD
CSLC = CNT_SZ // NS
ACC_R = 8448
ZR = ACC_R // NS

def _vgather(x, idx):
    return lax.gather(
        x, idx[:, None],
        lax.GatherDimensionNumbers(
            offset_dims=(), collapsed_slice_dims=(0,), start_index_map=(0,)),
        slice_sizes=(1,),
        mode=lax.GatherScatterMode.PROMISE_IN_BOUNDS)


_MESH = plsc.VectorSubcoreMesh(
    core_axis_name="c", subcore_axis_name="s", num_cores=NC, num_subcores=NS)


def _stage_a_body(srcp, dstp, xflat, cnt_out, fpk_out, nb_out,
                  sdh, ddh, cls, idx2, tgt, pk, zf, nbst, cnt_acc, fp_sh, sem):
    c = lax.axis_index("c")
    s = lax.axis_index("s")
    w = c * NS + s
    fbase = s * FCAP

    def _zf_zero(i, _):
        zf[pl.ds(i * 16, 16)] = jnp.zeros((16,), jnp.float32)
        return 0
    lax.fori_loop(0, CSLC // 16, _zf_zero, 0)
    pltpu.sync_copy(zf, cnt_acc.at[pl.ds(s * CSLC, CSLC)])

    def _ones(i, _):
        zf[pl.ds(i * 16, 16)] = jnp.ones((16,), jnp.float32)
        return 0
    lax.fori_loop(0, 8, _ones, 0)
    plsc.subcore_barrier()

    def _half(h, cur):
        pltpu.sync_copy(srcp.at[w, h], sdh)
        pltpu.sync_copy(dstp.at[w, h], ddh)

        def _gs(r, _):
            pltpu.make_async_copy(xflat.at[sdh.at[r]], cls.at[r], sem).start()
            return 0
        lax.fori_loop(0, HR, _gs, 0)

        def _gw(r, _):
            pltpu.make_async_copy(xflat.at[sdh.at[0]], cls.at[0], sem).wait()
            return 0
        lax.fori_loop(0, HR, _gw, 0)

        def _flt(i, cur):
            r = i >> 3
            col = (i & 7) * 16
            s16 = sdh[r, pl.ds(col, 16)]
            d16 = ddh[r, pl.ds(col, 16)]
            c16 = cls[r, pl.ds(col, 16)]
            idx2[r, pl.ds(col, 16)] = d16 * 4 + c16
            m = d16 < H
            packed = s16 * (2 * H) + jnp.where(m, d16, 0)
            lane = lax.iota(jnp.int32, 16)
            pos = jnp.where(m, 1, 0).astype(jnp.int32)
            for k in (1, 2, 4, 8):
                sh = _vgather(pos, jnp.maximum(lane - k, 0))
                pos = pos + jnp.where(lane >= k, sh, 0)
            tgt[r, pl.ds(col, 16)] = fbase + jnp.where(
                m, cur + pos - 1, FCAP - 1)
            pk[r, pl.ds(col, 16)] = packed
            pcv = _vgather(pos, jnp.full((16,), 15, jnp.int32))
            return cur + pcv
        cur = lax.fori_loop(0, HR * 8, _flt, cur)

        def _ss(r, _):
            pltpu.make_async_copy(
                zf.at[pl.ds(0, 128)], cnt_acc.at[idx2.at[r]], sem
            ).start(add=True)
            return 0
        lax.fori_loop(0, HR, _ss, 0)

        def _sw(r, _):
            pltpu.make_async_copy(
                zf.at[pl.ds(0, 128)], cnt_acc.at[idx2.at[0]], sem).wait()
            return 0
        lax.fori_loop(0, HR, _sw, 0)

        def _fsc(r, _):
            pltpu.make_async_copy(
                pk.at[r], fp_sh.at[tgt.at[r]], sem).start()
            return 0
        lax.fori_loop(0, HR, _fsc, 0)

        def _fsw(r, _):
            pltpu.make_async_copy(pk.at[0], fp_sh.at[tgt.at[0]], sem).wait()
            return 0
        lax.fori_loop(0, HR, _fsw, 0)
        return cur

    lane16 = lax.iota(jnp.int32, 16)
    cur = _half(0, jnp.zeros((16,), jnp.int32))
    cur = _half(1, cur)

    for k in range(8):
        tgt[0, pl.ds(k * 16, 16)] = fbase + cur + lane16 + k * 16
        pk[0, pl.ds(k * 16, 16)] = jnp.full((16,), H, jnp.int32)
    pltpu.sync_copy(pk.at[0], fp_sh.at[tgt.at[0]])

    nb128 = lax.shift_right_logical(cur + 127, 7)

    for k in range(8):
        nbst[pl.ds(k * 16, 16)] = nb128
    pltpu.sync_copy(nbst, nb_out.at[w])
    pltpu.sync_copy(fp_sh.at[pl.ds(fbase, FCAP)], fpk_out.at[w])

    plsc.subcore_barrier()
    pltpu.sync_copy(cnt_acc.at[pl.ds(s * CSLC, CSLC)],
                    cnt_out.at[c, pl.ds(s * CSLC, CSLC)])


_stage_a = functools.partial(
    pl.kernel,
    out_type=(
        jax.ShapeDtypeStruct((NC, CNT_SZ), jnp.float32),
        jax.ShapeDtypeStruct((NW, FCAP), jnp.int32),
        jax.ShapeDtypeStruct((NW, 128), jnp.int32),
    ),
    mesh=_MESH,
    scratch_types=(
        pltpu.VMEM((HR, 128), jnp.int32),
        pltpu.VMEM((HR, 128), jnp.int32),
        pltpu.VMEM((HR, 128), jnp.int32),
        pltpu.VMEM((HR, 128), jnp.int32),
        pltpu.VMEM((HR, 128), jnp.int32),
        pltpu.VMEM((HR, 128), jnp.int32),
        pltpu.VMEM((CSLC,), jnp.float32),
        pltpu.VMEM((128,), jnp.int32),
        pltpu.VMEM_SHARED((CNT_SZ,), jnp.float32),
        pltpu.VMEM_SHARED((NS * FCAP,), jnp.int32),
        pltpu.SemaphoreType.DMA,
    ),
)(_stage_a_body)


def _stage_c_body(z1, fpk, nb, s2_out, fp_v, fsr, fdr, rows, nbv, acc, sem):
    c = lax.axis_index("c")
    s = lax.axis_index("s")
    w = c * NS + s

    def _rz_all(i, _):
        r = i >> 3
        col = (i & 7) * 16
        rows[r, pl.ds(col, 16)] = jnp.zeros((16,), jnp.float32)
        return 0
    lax.fori_loop(0, 128 * 8, _rz_all, 0)
    base = s * ZR
    for k in range(4):
        pltpu.sync_copy(rows, acc.at[pl.ds(base + k * 128, 128)])
    pltpu.sync_copy(rows.at[pl.ds(0, 16)], acc.at[pl.ds(base + 512, 16)])
    plsc.subcore_barrier()

    pltpu.sync_copy(fpk.at[w], fp_v)
    pltpu.sync_copy(nb.at[w, pl.ds(0, 16)], nbv)
    n128 = nbv[...][0]

    def _batch(b, _):
        for k in range(8):
            v = fp_v[pl.ds(b * 128 + k * 16, 16)]
            fsr[pl.ds(k * 16, 16)] = lax.shift_right_logical(v, 14)
            fdr[pl.ds(k * 16, 16)] = v & (2 * H - 1)
        g = pltpu.make_async_copy(z1.at[fsr], rows, sem)
        g.start()
        g.wait()
        pltpu.sync_copy(rows, acc.at[fdr], add=True)
        return 0
    lax.fori_loop(0, n128, _batch, 0)

    plsc.subcore_barrier()
    pltpu.sync_copy(acc.at[pl.ds(s * 512, 512)],
                    s2_out.at[c, pl.ds(s * 512, 512)])


_stage_c = functools.partial(
    pl.kernel,
    out_type=jax.ShapeDtypeStruct((NC, H, 128), jnp.float32),
    mesh=_MESH,
    scratch_types=(
        pltpu.VMEM((FCAP,), jnp.int32),
        pltpu.VMEM((128,), jnp.int32),
        pltpu.VMEM((128,), jnp.int32),
        pltpu.VMEM((128, 128), jnp.float32),
        pltpu.VMEM((16,), jnp.int32),
        pltpu.VMEM_SHARED((ACC_R, 128), jnp.float32),
        pltpu.SemaphoreType.DMA,
    ),
)(_stage_c_body)


def _stage_b_body(cnt_ref, x_ref, m1_ref, r1_ref, bl1_ref, out_ref):
    cnt = cnt_ref[0] + cnt_ref[1]
    deg = jnp.sum(cnt, axis=1, keepdims=True)
    invd = 1.0 / jnp.maximum(deg, 1.0)
    a = cnt * invd
    xv = x_ref[...]
    out = jnp.broadcast_to(bl1_ref[...], (BLK, D))
    for c in range(4):
        out = out + a[:, c:c + 1] * m1_ref[c:c + 1, :]
        out = out + jnp.where(xv == c, 1.0, 0.0) * r1_ref[c:c + 1, :]
    norm = jnp.sqrt(jnp.sum(out * out, axis=1, keepdims=True))
    out = out / jnp.maximum(norm, 1e-12)
    out_ref[...] = jnp.concatenate(
        [jnp.maximum(out, 0.0), jnp.zeros((BLK, 128 - D), jnp.float32)],
        axis=1)


def _stage_b(cnt2, x_pad, m1, r1, bl1):
    grid = (NPAD // BLK,)
    return pl.pallas_call(
        _stage_b_body,
        grid=grid,
        in_specs=[
            pl.BlockSpec((2, BLK, 4), lambda i: (0, i, 0)),
            pl.BlockSpec((BLK, 1), lambda i: (i, 0)),
            pl.BlockSpec((4, D), lambda i: (0, 0)),
            pl.BlockSpec((4, D), lambda i: (0, 0)),
            pl.BlockSpec((1, D), lambda i: (0, 0)),
        ],
        out_specs=pl.BlockSpec((BLK, 128), lambda i: (i, 0)),
        out_shape=jax.ShapeDtypeStruct((NPAD, 128), jnp.float32),
    )(cnt2, x_pad, m1, r1, bl1)


def _stage_d_body(s2_ref, cnt_ref, z1_ref, wl2_ref, wr2_ref, bl2_ref,
                  w1_ref, b1_ref, w2_ref, b2_ref, out_ref):
    s = s2_ref[0, :, :D] + s2_ref[1, :, :D]
    cnt = cnt_ref[0] + cnt_ref[1]
    deg = jnp.sum(cnt, axis=1, keepdims=True)
    aggr = s * (1.0 / jnp.maximum(deg, 1.0))
    z1s = z1_ref[:, :D]
    dn = (((1,), (1,)), ((), ()))
    out = (lax.dot_general(aggr, wl2_ref[...], dn,
                           preferred_element_type=jnp.float32)
           + lax.dot_general(z1s, wr2_ref[...], dn,
                             preferred_element_type=jnp.float32)
           + bl2_ref[...])
    norm = jnp.sqrt(jnp.sum(out * out, axis=1, keepdims=True))
    z2 = jnp.maximum(out / jnp.maximum(norm, 1e-12), 0.0)
    za = z2[:B]
    zb = z2[B:]
    w1a = w1_ref[:, :D]
    w1b = w1_ref[:, D:]
    h = (lax.dot_general(za, w1a, dn, preferred_element_type=jnp.float32)
         + lax.dot_general(zb, w1b, dn, preferred_element_type=jnp.float32)
         + b1_ref[...])
    h = jnp.maximum(h, 0.0)
    pred = jnp.sum(h * w2_ref[...], axis=1, keepdims=True) + b2_ref[...]
    out_ref[...] = pred


def _stage_d(s2, cnt2h, z1h, wl2, wr2, bl2, w1, b1, w2, b2):
    return pl.pallas_call(
        _stage_d_body,
        grid=(1,),
        in_specs=[
            pl.BlockSpec((2, H, 128), lambda i: (0, 0, 0)),
            pl.BlockSpec((2, H, 4), lambda i: (0, 0, 0)),
            pl.BlockSpec((H, 128), lambda i: (0, 0)),
            pl.BlockSpec((D, D), lambda i: (0, 0)),
            pl.BlockSpec((D, D), lambda i: (0, 0)),
            pl.BlockSpec((1, D), lambda i: (0, 0)),
            pl.BlockSpec((D, 2 * D), lambda i: (0, 0)),
            pl.BlockSpec((1, D), lambda i: (0, 0)),
            pl.BlockSpec((1, D), lambda i: (0, 0)),
            pl.BlockSpec((1, 1), lambda i: (0, 0)),
        ],
        out_specs=pl.BlockSpec((B, 1), lambda i: (0, 0)),
        out_shape=jax.ShapeDtypeStruct((B, 1), jnp.float32),
    )(s2, cnt2h, z1h, wl2, wr2, bl2, w1, b1, w2, b2)


def kernel(x, edge_index, edge_label, node_emb,
           Wl1, bl1, Wr1, Wl2, bl2, Wr2, W1, b1, W2, b2):
    m1 = node_emb @ Wl1.T
    r1 = node_emb @ Wr1.T
    x_pad = jnp.pad(x, ((0, NPAD - N), (0, 0)))
    xflat = x[:, 0]

    padn = EPAD - E
    src_p = jnp.concatenate(
        [edge_index[0], jnp.zeros((padn,), jnp.int32)]).reshape(NW, 2, HR, 128)
    dst_p = jnp.concatenate(
        [edge_index[1],
         jnp.full((padn,), NPAD - 1, jnp.int32)]).reshape(NW, 2, HR, 128)

    cnt_flat, fpk, nb = _stage_a(src_p, dst_p, xflat)
    cnt2 = cnt_flat.reshape(NC, NPAD, 4)

    z1 = _stage_b(cnt2, x_pad, m1, r1, bl1.reshape(1, D))
    s2 = _stage_c(z1, fpk, nb)
    pred = _stage_d(s2, cnt2, z1,
                    Wl2, Wr2, bl2.reshape(1, D),
                    W1, b1.reshape(1, D), W2.reshape(1, D),
                    b2.reshape(1, 1))
    return (pred, edge_label)


# speedup vs baseline: 16.4065x; 1.1085x over previous
import functools
import jax
import jax.numpy as jnp
from jax import lax
from jax.experimental import pallas as pl
from jax.experimental.pallas import tpu as pltpu
from jax.experimental.pallas import tpu_sc as plsc

N = 50000
E = 800000
D = 64
B = 4096
H = 2 * B
NPAD = 50176
BLK = 512
NC = 2
NS = 16
NW = NC * NS
EC = 25088
EPAD = NW * EC
HR = 98
FCAP = EC + 128
CNT_SZ = 8 * NPAD
CSLC = CNT_SZ // NS
ZSLC = CSLC // 4
ACC_R = 8448
ZR = ACC_R // NS

def _vgather(x, idx):
    return lax.gather(
        x, idx[:, None],
        lax.GatherDimensionNumbers(
            offset_dims=(), collapsed_slice_dims=(0,), start_index_map=(0,)),
        slice_sizes=(1,),
        mode=lax.GatherScatterMode.PROMISE_IN_BOUNDS)


_MESH = plsc.VectorSubcoreMesh(
    core_axis_name="c", subcore_axis_name="s", num_cores=NC, num_subcores=NS)


def _stage_a_body(srcp, dstp, xflat, xp2, cnt_out, fpk_out, nb_out,
                  sdh, ddh, cls, tgt, pk, zf, xbuf, nbst, cnt_acc, fp_sh, sem):
    c = lax.axis_index("c")
    s = lax.axis_index("s")
    w = c * NS + s
    fbase = s * FCAP

    def _zf_zero(i, _):
        zf[pl.ds(i * 16, 16)] = jnp.zeros((16,), jnp.float32)
        return 0
    lax.fori_loop(0, ZSLC // 16, _zf_zero, 0)
    for q in range(4):
        pltpu.sync_copy(zf, cnt_acc.at[pl.ds(s * CSLC + q * ZSLC, ZSLC)])

    def _ones(i, _):
        zf[pl.ds(i * 16, 16)] = jnp.ones((16,), jnp.float32)
        return 0
    lax.fori_loop(0, 8, _ones, 0)
    plsc.subcore_barrier()

    def _half(h, cur):
        pltpu.sync_copy(srcp.at[w, h], sdh)
        pltpu.sync_copy(dstp.at[w, h], ddh)

        def _gs(r, _):
            pltpu.make_async_copy(xflat.at[sdh.at[r]], cls.at[r], sem).start()
            return 0
        lax.fori_loop(0, HR, _gs, 0)

        def _gw(r, _):
            pltpu.make_async_copy(xflat.at[sdh.at[0]], cls.at[0], sem).wait()
            return 0
        lax.fori_loop(0, HR, _gw, 0)

        def _flt(i, cur):
            r = i >> 3
            col = (i & 7) * 16
            s16 = sdh[r, pl.ds(col, 16)]
            d16 = ddh[r, pl.ds(col, 16)]
            m = d16 < H
            packed = s16 * (2 * H) + jnp.where(m, d16, 0)
            lane = lax.iota(jnp.int32, 16)
            pos = jnp.where(m, 1, 0).astype(jnp.int32)
            for k in (1, 2, 4, 8):
                sh = _vgather(pos, jnp.maximum(lane - k, 0))
                pos = pos + jnp.where(lane >= k, sh, 0)
            tgt[r, pl.ds(col, 16)] = fbase + jnp.where(
                m, cur + pos - 1, FCAP - 1)
            pk[r, pl.ds(col, 16)] = packed
            pcv = _vgather(pos, jnp.full((16,), 15, jnp.int32))
            return cur + pcv
        cur = lax.fori_loop(0, HR * 8, _flt, cur)

        def _fsc(r, _):
            pltpu.make_async_copy(
                pk.at[r], fp_sh.at[tgt.at[r]], sem).start()
            return 0
        lax.fori_loop(0, HR, _fsc, 0)

        def _fsw(r, _):
            pltpu.make_async_copy(pk.at[0], fp_sh.at[tgt.at[0]], sem).wait()
            return 0
        lax.fori_loop(0, HR, _fsw, 0)

        def _hix(i, _):
            r = i >> 3
            col = (i & 7) * 16
            d16 = ddh[r, pl.ds(col, 16)]
            c16 = cls[r, pl.ds(col, 16)]
            tgt[r, pl.ds(col, 16)] = d16 * 8 + c16
            return 0
        lax.fori_loop(0, HR * 8, _hix, 0)

        def _ss(r, _):
            pltpu.make_async_copy(
                zf.at[pl.ds(0, 128)], cnt_acc.at[tgt.at[r]], sem
            ).start(add=True)
            return 0
        lax.fori_loop(0, HR, _ss, 0)

        def _sw(r, _):
            pltpu.make_async_copy(
                zf.at[pl.ds(0, 128)], cnt_acc.at[tgt.at[0]], sem).wait()
            return 0
        lax.fori_loop(0, HR, _sw, 0)
        return cur

    lane16 = lax.iota(jnp.int32, 16)
    cur = _half(0, jnp.zeros((16,), jnp.int32))
    cur = _half(1, cur)

    nr = jnp.where(w < 8, 13, 12)
    base = w * 12 + jnp.minimum(w, 8)
    pltpu.sync_copy(xp2.at[pl.ds(base * 128, 13 * 128)], xbuf)

    def _nhx(i, _):
        r = i >> 3
        col = (i & 7) * 16
        node = base * 128 + i * 16 + lane16
        xv = xbuf[pl.ds(i * 16, 16)]
        tgt[r, pl.ds(col, 16)] = node * 8 + 4 + xv
        return 0
    lax.fori_loop(0, nr * 8, _nhx, 0)

    def _nsc(r, _):
        pltpu.make_async_copy(
            zf.at[pl.ds(0, 128)], cnt_acc.at[tgt.at[r]], sem).start(add=True)
        return 0
    lax.fori_loop(0, nr, _nsc, 0)

    def _nsw(r, _):
        pltpu.make_async_copy(
            zf.at[pl.ds(0, 128)], cnt_acc.at[tgt.at[0]], sem).wait()
        return 0
    lax.fori_loop(0, nr, _nsw, 0)

    for k in range(8):
        tgt[0, pl.ds(k * 16, 16)] = fbase + cur + lane16 + k * 16
        pk[0, pl.ds(k * 16, 16)] = jnp.full((16,), H, jnp.int32)
    pltpu.sync_copy(pk.at[0], fp_sh.at[tgt.at[0]])

    nb128 = lax.shift_right_logical(cur + 127, 7)

    for k in range(8):
        nbst[pl.ds(k * 16, 16)] = nb128
    pltpu.sync_copy(nbst, nb_out.at[w])
    pltpu.sync_copy(fp_sh.at[pl.ds(fbase, FCAP)], fpk_out.at[w])

    plsc.subcore_barrier()
    pltpu.sync_copy(cnt_acc.at[pl.ds(s * CSLC, CSLC)],
                    cnt_out.at[c, pl.ds(s * CSLC, CSLC)])


_stage_a = functools.partial(
    pl.kernel,
    out_type=(
        jax.ShapeDtypeStruct((NC, CNT_SZ), jnp.float32),
        jax.ShapeDtypeStruct((NW, FCAP), jnp.int32),
        jax.ShapeDtypeStruct((NW, 128), jnp.int32),
    ),
    mesh=_MESH,
    scratch_types=(
        pltpu.VMEM((HR, 128), jnp.int32),
        pltpu.VMEM((HR, 128), jnp.int32),
        pltpu.VMEM((HR, 128), jnp.int32),
        pltpu.VMEM((HR, 128), jnp.int32),
        pltpu.VMEM((HR, 128), jnp.int32),
        pltpu.VMEM((ZSLC,), jnp.float32),
        pltpu.VMEM((13 * 128,), jnp.int32),
        pltpu.VMEM((128,), jnp.int32),
        pltpu.VMEM_SHARED((CNT_SZ,), jnp.float32),
        pltpu.VMEM_SHARED((NS * FCAP,), jnp.int32),
        pltpu.SemaphoreType.DMA,
    ),
)(_stage_a_body)


def _stage_c_body(z1, fpk, nb, s2_out, fp_v, fsr, fdr, rows, nbv, acc, sem):
    c = lax.axis_index("c")
    s = lax.axis_index("s")
    w = c * NS + s

    def _rz_all(i, _):
        r = i >> 3
        col = (i & 7) * 16
        rows[r, pl.ds(col, 16)] = jnp.zeros((16,), jnp.float32)
        return 0
    lax.fori_loop(0, 128 * 8, _rz_all, 0)
    base = s * ZR
    for k in range(4):
        pltpu.sync_copy(rows, acc.at[pl.ds(base + k * 128, 128)])
    pltpu.sync_copy(rows.at[pl.ds(0, 16)], acc.at[pl.ds(base + 512, 16)])
    plsc.subcore_barrier()

    pltpu.sync_copy(fpk.at[w], fp_v)
    pltpu.sync_copy(nb.at[w, pl.ds(0, 16)], nbv)
    n128 = nbv[...][0]

    def _batch(b, _):
        for k in range(8):
            v = fp_v[pl.ds(b * 128 + k * 16, 16)]
            fsr[pl.ds(k * 16, 16)] = lax.shift_right_logical(v, 14)
            fdr[pl.ds(k * 16, 16)] = v & (2 * H - 1)
        g = pltpu.make_async_copy(z1.at[fsr], rows, sem)
        g.start()
        g.wait()
        pltpu.sync_copy(rows, acc.at[fdr], add=True)
        return 0
    lax.fori_loop(0, n128, _batch, 0)

    plsc.subcore_barrier()
    pltpu.sync_copy(acc.at[pl.ds(s * 512, 512)],
                    s2_out.at[c, pl.ds(s * 512, 512)])


_stage_c = functools.partial(
    pl.kernel,
    out_type=jax.ShapeDtypeStruct((NC, H, 128), jnp.float32),
    mesh=_MESH,
    scratch_types=(
        pltpu.VMEM((FCAP,), jnp.int32),
        pltpu.VMEM((128,), jnp.int32),
        pltpu.VMEM((128,), jnp.int32),
        pltpu.VMEM((128, 128), jnp.float32),
        pltpu.VMEM((16,), jnp.int32),
        pltpu.VMEM_SHARED((ACC_R, 128), jnp.float32),
        pltpu.SemaphoreType.DMA,
    ),
)(_stage_c_body)


def _stage_b_body(cnt_ref, mr_ref, bl1_ref, out_ref):
    cnt = cnt_ref[0] + cnt_ref[1]
    deg = jnp.sum(cnt[:, :4], axis=1, keepdims=True)
    invd = 1.0 / jnp.maximum(deg, 1.0)
    col = lax.broadcasted_iota(jnp.int32, (BLK, 8), 1)
    a = cnt * jnp.where(col < 4, invd, 1.0)
    out = lax.dot_general(a, mr_ref[...], (((1,), (0,)), ((), ())),
                          preferred_element_type=jnp.float32) + bl1_ref[...]
    norm = jnp.sqrt(jnp.sum(out * out, axis=1, keepdims=True))
    out = out / jnp.maximum(norm, 1e-12)
    out_ref[...] = jnp.concatenate(
        [jnp.maximum(out, 0.0), jnp.zeros((BLK, 128 - D), jnp.float32)],
        axis=1)


def _stage_b(cnt2, m1r1, bl1):
    grid = (NPAD // BLK,)
    return pl.pallas_call(
        _stage_b_body,
        grid=grid,
        in_specs=[
            pl.BlockSpec((2, BLK, 8), lambda i: (0, i, 0)),
            pl.BlockSpec((8, D), lambda i: (0, 0)),
            pl.BlockSpec((1, D), lambda i: (0, 0)),
        ],
        out_specs=pl.BlockSpec((BLK, 128), lambda i: (i, 0)),
        out_shape=jax.ShapeDtypeStruct((NPAD, 128), jnp.float32),
    )(cnt2, m1r1, bl1)


def _stage_d_body(s2_ref, cnt_ref, z1_ref, wl2_ref, wr2_ref, bl2_ref,
                  w1_ref, b1_ref, w2_ref, b2_ref, out_ref):
    s = s2_ref[0, :, :D] + s2_ref[1, :, :D]
    cnt = cnt_ref[0] + cnt_ref[1]
    deg = jnp.sum(cnt[:, :4], axis=1, keepdims=True)
    aggr = s * (1.0 / jnp.maximum(deg, 1.0))
    z1s = z1_ref[:, :D]
    dn = (((1,), (1,)), ((), ()))
    out = (lax.dot_general(aggr, wl2_ref[...], dn,
                           preferred_element_type=jnp.float32)
           + lax.dot_general(z1s, wr2_ref[...], dn,
                             preferred_element_type=jnp.float32)
           + bl2_ref[...])
    norm = jnp.sqrt(jnp.sum(out * out, axis=1, keepdims=True))
    z2 = jnp.maximum(out / jnp.maximum(norm, 1e-12), 0.0)
    za = z2[:B]
    zb = z2[B:]
    w1a = w1_ref[:, :D]
    w1b = w1_ref[:, D:]
    h = (lax.dot_general(za, w1a, dn, preferred_element_type=jnp.float32)
         + lax.dot_general(zb, w1b, dn, preferred_element_type=jnp.float32)
         + b1_ref[...])
    h = jnp.maximum(h, 0.0)
    pred = jnp.sum(h * w2_ref[...], axis=1, keepdims=True) + b2_ref[...]
    out_ref[...] = pred


def _stage_d(s2, cnt2h, z1h, wl2, wr2, bl2, w1, b1, w2, b2):
    return pl.pallas_call(
        _stage_d_body,
        grid=(1,),
        in_specs=[
            pl.BlockSpec((2, H, 128), lambda i: (0, 0, 0)),
            pl.BlockSpec((2, H, 8), lambda i: (0, 0, 0)),
            pl.BlockSpec((H, 128), lambda i: (0, 0)),
            pl.BlockSpec((D, D), lambda i: (0, 0)),
            pl.BlockSpec((D, D), lambda i: (0, 0)),
            pl.BlockSpec((1, D), lambda i: (0, 0)),
            pl.BlockSpec((D, 2 * D), lambda i: (0, 0)),
            pl.BlockSpec((1, D), lambda i: (0, 0)),
            pl.BlockSpec((1, D), lambda i: (0, 0)),
            pl.BlockSpec((1, 1), lambda i: (0, 0)),
        ],
        out_specs=pl.BlockSpec((B, 1), lambda i: (0, 0)),
        out_shape=jax.ShapeDtypeStruct((B, 1), jnp.float32),
    )(s2, cnt2h, z1h, wl2, wr2, bl2, w1, b1, w2, b2)


def kernel(x, edge_index, edge_label, node_emb,
           Wl1, bl1, Wr1, Wl2, bl2, Wr2, W1, b1, W2, b2):
    m1r1 = jnp.concatenate([node_emb @ Wl1.T, node_emb @ Wr1.T])
    xflat = x[:, 0]
    xp2 = jnp.pad(xflat, (0, NPAD + 128 - N))

    padn = EPAD - E
    src_p = jnp.concatenate(
        [edge_index[0], jnp.zeros((padn,), jnp.int32)]).reshape(NW, 2, HR, 128)
    dst_p = jnp.concatenate(
        [edge_index[1],
         jnp.full((padn,), NPAD - 1, jnp.int32)]).reshape(NW, 2, HR, 128)

    cnt_flat, fpk, nb = _stage_a(src_p, dst_p, xflat, xp2)
    cnt2 = cnt_flat.reshape(NC, NPAD, 8)

    z1 = _stage_b(cnt2, m1r1, bl1.reshape(1, D))
    s2 = _stage_c(z1, fpk, nb)
    pred = _stage_d(s2, cnt2, z1,
                    Wl2, Wr2, bl2.reshape(1, D),
                    W1, b1.reshape(1, D), W2.reshape(1, D),
                    b2.reshape(1, 1))
    return (pred, edge_label)


# speedup vs baseline: 17.0290x; 1.0379x over previous
import functools
import jax
import jax.numpy as jnp
from jax import lax
from jax.experimental import pallas as pl
from jax.experimental.pallas import tpu as pltpu
from jax.experimental.pallas import tpu_sc as plsc

N = 50000
E = 800000
D = 64
B = 4096
H = 2 * B
NPAD = 50176
BLK = 512
NC = 2
NS = 16
NW = NC * NS
EC = 25088
EPAD = NW * EC
HR = 98
FCAP = EC + 128
CNT_SZ = 8 * NPAD
CSLC = CNT_SZ // NS
ZSLC = CSLC // 4
ACC_R = 8448
ZR = ACC_R // NS

def _vgather(x, idx):
    return lax.gather(
        x, idx[:, None],
        lax.GatherDimensionNumbers(
            offset_dims=(), collapsed_slice_dims=(0,), start_index_map=(0,)),
        slice_sizes=(1,),
        mode=lax.GatherScatterMode.PROMISE_IN_BOUNDS)


_MESH = plsc.VectorSubcoreMesh(
    core_axis_name="c", subcore_axis_name="s", num_cores=NC, num_subcores=NS)


def _stage_a_body(srcp, dstp, xflat, xp2, cnt_out, fpk_out, nb_out,
                  sdh, ddh, cls, tgt, pk, zf, xbuf, nbst, cnt_acc, fp_sh, sem):
    c = lax.axis_index("c")
    s = lax.axis_index("s")
    w = c * NS + s
    fbase = s * FCAP

    def _zf_zero(i, _):
        zf[pl.ds(i * 16, 16)] = jnp.zeros((16,), jnp.float32)
        return 0
    lax.fori_loop(0, ZSLC // 16, _zf_zero, 0)
    for q in range(4):
        pltpu.sync_copy(zf, cnt_acc.at[pl.ds(s * CSLC + q * ZSLC, ZSLC)])

    def _ones(i, _):
        zf[pl.ds(i * 16, 16)] = jnp.ones((16,), jnp.float32)
        return 0
    lax.fori_loop(0, 8, _ones, 0)
    plsc.subcore_barrier()

    def _half(h, cur):
        pltpu.sync_copy(srcp.at[w, h], sdh)
        pltpu.sync_copy(dstp.at[w, h], ddh)

        def _gs(r, _):
            pltpu.make_async_copy(xflat.at[sdh.at[r]], cls.at[r], sem).start()
            return 0
        lax.fori_loop(0, HR, _gs, 0)

        def _gw(r, _):
            pltpu.make_async_copy(xflat.at[sdh.at[0]], cls.at[0], sem).wait()
            return 0
        lax.fori_loop(0, HR, _gw, 0)

        def _flt(i, cur):
            r = i >> 3
            col = (i & 7) * 16
            s16 = sdh[r, pl.ds(col, 16)]
            d16 = ddh[r, pl.ds(col, 16)]
            m = d16 < H
            packed = s16 * (2 * H) + jnp.where(m, d16, 0)
            lane = lax.iota(jnp.int32, 16)
            pos = jnp.where(m, 1, 0).astype(jnp.int32)
            for k in (1, 2, 4, 8):
                sh = _vgather(pos, jnp.maximum(lane - k, 0))
                pos = pos + jnp.where(lane >= k, sh, 0)
            tgt[r, pl.ds(col, 16)] = fbase + jnp.where(
                m, cur + pos - 1, FCAP - 1)
            pk[r, pl.ds(col, 16)] = packed
            pcv = _vgather(pos, jnp.full((16,), 15, jnp.int32))
            return cur + pcv
        cur = lax.fori_loop(0, HR * 8, _flt, cur)

        def _fsc(r, _):
            pltpu.make_async_copy(
                pk.at[r], fp_sh.at[tgt.at[r]], sem).start()
            return 0
        lax.fori_loop(0, HR, _fsc, 0)

        def _fsw(r, _):
            pltpu.make_async_copy(pk.at[0], fp_sh.at[tgt.at[0]], sem).wait()
            return 0
        lax.fori_loop(0, HR, _fsw, 0)

        def _hix(i, _):
            r = i >> 3
            col = (i & 7) * 16
            d16 = ddh[r, pl.ds(col, 16)]
            c16 = cls[r, pl.ds(col, 16)]
            tgt[r, pl.ds(col, 16)] = d16 * 8 + c16
            return 0
        lax.fori_loop(0, HR * 8, _hix, 0)

        def _ss(r, _):
            pltpu.make_async_copy(
                zf.at[pl.ds(0, 128)], cnt_acc.at[tgt.at[r]], sem
            ).start(add=True)
            return 0
        lax.fori_loop(0, HR, _ss, 0)

        def _sw(r, _):
            pltpu.make_async_copy(
                zf.at[pl.ds(0, 128)], cnt_acc.at[tgt.at[0]], sem).wait()
            return 0
        lax.fori_loop(0, HR, _sw, 0)
        return cur

    lane16 = lax.iota(jnp.int32, 16)
    cur = _half(0, jnp.zeros((16,), jnp.int32))
    cur = _half(1, cur)

    nr = jnp.where(w < 8, 13, 12)
    base = w * 12 + jnp.minimum(w, 8)
    pltpu.sync_copy(xp2.at[pl.ds(base * 128, 13 * 128)], xbuf)

    def _nhx(i, _):
        r = i >> 3
        col = (i & 7) * 16
        node = base * 128 + i * 16 + lane16
        xv = xbuf[pl.ds(i * 16, 16)]
        tgt[r, pl.ds(col, 16)] = node * 8 + 4 + xv
        return 0
    lax.fori_loop(0, nr * 8, _nhx, 0)

    def _nsc(r, _):
        pltpu.make_async_copy(
            zf.at[pl.ds(0, 128)], cnt_acc.at[tgt.at[r]], sem).start(add=True)
        return 0
    lax.fori_loop(0, nr, _nsc, 0)

    def _nsw(r, _):
        pltpu.make_async_copy(
            zf.at[pl.ds(0, 128)], cnt_acc.at[tgt.at[0]], sem).wait()
        return 0
    lax.fori_loop(0, nr, _nsw, 0)

    for k in range(8):
        tgt[0, pl.ds(k * 16, 16)] = fbase + cur + lane16 + k * 16
        pk[0, pl.ds(k * 16, 16)] = jnp.full((16,), H, jnp.int32)
    pltpu.sync_copy(pk.at[0], fp_sh.at[tgt.at[0]])

    nb128 = lax.shift_right_logical(cur + 127, 7)

    for k in range(8):
        nbst[pl.ds(k * 16, 16)] = nb128
    pltpu.sync_copy(nbst, nb_out.at[w])
    pltpu.sync_copy(fp_sh.at[pl.ds(fbase, FCAP)], fpk_out.at[w])

    plsc.subcore_barrier()
    pltpu.sync_copy(cnt_acc.at[pl.ds(s * CSLC, CSLC)],
                    cnt_out.at[c, pl.ds(s * CSLC, CSLC)])


_stage_a = functools.partial(
    pl.kernel,
    out_type=(
        jax.ShapeDtypeStruct((NC, CNT_SZ), jnp.float32),
        jax.ShapeDtypeStruct((NW, FCAP), jnp.int32),
        jax.ShapeDtypeStruct((NW, 128), jnp.int32),
    ),
    mesh=_MESH,
    scratch_types=(
        pltpu.VMEM((HR, 128), jnp.int32),
        pltpu.VMEM((HR, 128), jnp.int32),
        pltpu.VMEM((HR, 128), jnp.int32),
        pltpu.VMEM((HR, 128), jnp.int32),
        pltpu.VMEM((HR, 128), jnp.int32),
        pltpu.VMEM((ZSLC,), jnp.float32),
        pltpu.VMEM((13 * 128,), jnp.int32),
        pltpu.VMEM((128,), jnp.int32),
        pltpu.VMEM_SHARED((CNT_SZ,), jnp.float32),
        pltpu.VMEM_SHARED((NS * FCAP,), jnp.int32),
        pltpu.SemaphoreType.DMA,
    ),
)(_stage_a_body)


def _stage_c_body(z1, fpk, nb, s2_out, fp_v, fsr0, fdr0, fsr1, fdr1,
                  rows0, rows1, nbv, acc, sem, addsem):
    c = lax.axis_index("c")
    s = lax.axis_index("s")
    w = c * NS + s

    def _rz_all(i, _):
        r = i >> 3
        col = (i & 7) * 16
        rows0[r, pl.ds(col, 16)] = jnp.zeros((16,), jnp.float32)
        return 0
    lax.fori_loop(0, 128 * 8, _rz_all, 0)
    base = s * ZR
    for k in range(4):
        pltpu.sync_copy(rows0, acc.at[pl.ds(base + k * 128, 128)])
    pltpu.sync_copy(rows0.at[pl.ds(0, 16)], acc.at[pl.ds(base + 512, 16)])
    plsc.subcore_barrier()

    pltpu.sync_copy(fpk.at[w], fp_v)
    pltpu.sync_copy(nb.at[w, pl.ds(0, 16)], nbv)
    n128 = nbv[...][0]

    bufs = ((fsr0, fdr0, rows0), (fsr1, fdr1, rows1))

    def _batch2(gg, _):
        for p in range(2):
            b = gg * 2 + p
            fsr, fdr, rows = bufs[p]

            @pl.when(b < n128)
            def _():
                for k in range(8):
                    v = fp_v[pl.ds(b * 128 + k * 16, 16)]
                    fsr[pl.ds(k * 16, 16)] = lax.shift_right_logical(v, 14)
                    fdr[pl.ds(k * 16, 16)] = v & (2 * H - 1)

                @pl.when(b >= 2)
                def _():
                    pltpu.make_async_copy(rows, acc.at[fdr], addsem).wait()

                g = pltpu.make_async_copy(z1.at[fsr], rows, sem)
                g.start()
                g.wait()
                pltpu.make_async_copy(rows, acc.at[fdr], addsem).start(
                    add=True)
        return 0
    lax.fori_loop(0, (n128 + 1) // 2, _batch2, 0)

    @pl.when(n128 >= 1)
    def _():
        pltpu.make_async_copy(rows0, acc.at[fdr0], addsem).wait()

    @pl.when(n128 >= 2)
    def _():
        pltpu.make_async_copy(rows0, acc.at[fdr0], addsem).wait()

    plsc.subcore_barrier()
    pltpu.sync_copy(acc.at[pl.ds(s * 512, 512)],
                    s2_out.at[c, pl.ds(s * 512, 512)])


_stage_c = functools.partial(
    pl.kernel,
    out_type=jax.ShapeDtypeStruct((NC, H, 128), jnp.float32),
    mesh=_MESH,
    scratch_types=(
        pltpu.VMEM((FCAP,), jnp.int32),
        pltpu.VMEM((128,), jnp.int32),
        pltpu.VMEM((128,), jnp.int32),
        pltpu.VMEM((128,), jnp.int32),
        pltpu.VMEM((128,), jnp.int32),
        pltpu.VMEM((128, 128), jnp.float32),
        pltpu.VMEM((128, 128), jnp.float32),
        pltpu.VMEM((16,), jnp.int32),
        pltpu.VMEM_SHARED((ACC_R, 128), jnp.float32),
        pltpu.SemaphoreType.DMA,
        pltpu.SemaphoreType.DMA,
    ),
)(_stage_c_body)


def _stage_b_body(cnt_ref, mr_ref, bl1_ref, out_ref):
    cnt = cnt_ref[0] + cnt_ref[1]
    deg = jnp.sum(cnt[:, :4], axis=1, keepdims=True)
    invd = 1.0 / jnp.maximum(deg, 1.0)
    col = lax.broadcasted_iota(jnp.int32, (BLK, 8), 1)
    a = cnt * jnp.where(col < 4, invd, 1.0)
    out = lax.dot_general(a, mr_ref[...], (((1,), (0,)), ((), ())),
                          preferred_element_type=jnp.float32) + bl1_ref[...]
    norm = jnp.sqrt(jnp.sum(out * out, axis=1, keepdims=True))
    out = out / jnp.maximum(norm, 1e-12)
    out_ref[...] = jnp.concatenate(
        [jnp.maximum(out, 0.0), jnp.zeros((BLK, 128 - D), jnp.float32)],
        axis=1)


def _stage_b(cnt2, m1r1, bl1):
    grid = (NPAD // BLK,)
    return pl.pallas_call(
        _stage_b_body,
        grid=grid,
        in_specs=[
            pl.BlockSpec((2, BLK, 8), lambda i: (0, i, 0)),
            pl.BlockSpec((8, D), lambda i: (0, 0)),
            pl.BlockSpec((1, D), lambda i: (0, 0)),
        ],
        out_specs=pl.BlockSpec((BLK, 128), lambda i: (i, 0)),
        out_shape=jax.ShapeDtypeStruct((NPAD, 128), jnp.float32),
    )(cnt2, m1r1, bl1)


def _stage_d_body(s2_ref, cnt_ref, z1_ref, wl2_ref, wr2_ref, bl2_ref,
                  w1_ref, b1_ref, w2_ref, b2_ref, out_ref):
    s = s2_ref[0, :, :D] + s2_ref[1, :, :D]
    cnt = cnt_ref[0] + cnt_ref[1]
    deg = jnp.sum(cnt[:, :4], axis=1, keepdims=True)
    aggr = s * (1.0 / jnp.maximum(deg, 1.0))
    z1s = z1_ref[:, :D]
    dn = (((1,), (1,)), ((), ()))
    out = (lax.dot_general(aggr, wl2_ref[...], dn,
                           preferred_element_type=jnp.float32)
           + lax.dot_general(z1s, wr2_ref[...], dn,
                             preferred_element_type=jnp.float32)
           + bl2_ref[...])
    norm = jnp.sqrt(jnp.sum(out * out, axis=1, keepdims=True))
    z2 = jnp.maximum(out / jnp.maximum(norm, 1e-12), 0.0)
    za = z2[:B]
    zb = z2[B:]
    w1a = w1_ref[:, :D]
    w1b = w1_ref[:, D:]
    h = (lax.dot_general(za, w1a, dn, preferred_element_type=jnp.float32)
         + lax.dot_general(zb, w1b, dn, preferred_element_type=jnp.float32)
         + b1_ref[...])
    h = jnp.maximum(h, 0.0)
    pred = jnp.sum(h * w2_ref[...], axis=1, keepdims=True) + b2_ref[...]
    out_ref[...] = pred


def _stage_d(s2, cnt2h, z1h, wl2, wr2, bl2, w1, b1, w2, b2):
    return pl.pallas_call(
        _stage_d_body,
        grid=(1,),
        in_specs=[
            pl.BlockSpec((2, H, 128), lambda i: (0, 0, 0)),
            pl.BlockSpec((2, H, 8), lambda i: (0, 0, 0)),
            pl.BlockSpec((H, 128), lambda i: (0, 0)),
            pl.BlockSpec((D, D), lambda i: (0, 0)),
            pl.BlockSpec((D, D), lambda i: (0, 0)),
            pl.BlockSpec((1, D), lambda i: (0, 0)),
            pl.BlockSpec((D, 2 * D), lambda i: (0, 0)),
            pl.BlockSpec((1, D), lambda i: (0, 0)),
            pl.BlockSpec((1, D), lambda i: (0, 0)),
            pl.BlockSpec((1, 1), lambda i: (0, 0)),
        ],
        out_specs=pl.BlockSpec((B, 1), lambda i: (0, 0)),
        out_shape=jax.ShapeDtypeStruct((B, 1), jnp.float32),
    )(s2, cnt2h, z1h, wl2, wr2, bl2, w1, b1, w2, b2)


def kernel(x, edge_index, edge_label, node_emb,
           Wl1, bl1, Wr1, Wl2, bl2, Wr2, W1, b1, W2, b2):
    m1r1 = jnp.concatenate([node_emb @ Wl1.T, node_emb @ Wr1.T])
    xflat = x[:, 0]
    xp2 = jnp.pad(xflat, (0, NPAD + 128 - N))

    padn = EPAD - E
    src_p = jnp.concatenate(
        [edge_index[0], jnp.zeros((padn,), jnp.int32)]).reshape(NW, 2, HR, 128)
    dst_p = jnp.concatenate(
        [edge_index[1],
         jnp.full((padn,), NPAD - 1, jnp.int32)]).reshape(NW, 2, HR, 128)

    cnt_flat, fpk, nb = _stage_a(src_p, dst_p, xflat, xp2)
    cnt2 = cnt_flat.reshape(NC, NPAD, 8)

    z1 = _stage_b(cnt2, m1r1, bl1.reshape(1, D))
    s2 = _stage_c(z1, fpk, nb)
    pred = _stage_d(s2, cnt2, z1,
                    Wl2, Wr2, bl2.reshape(1, D),
                    W1, b1.reshape(1, D), W2.reshape(1, D),
                    b2.reshape(1, 1))
    return (pred, edge_label)


# speedup vs baseline: 17.0378x; 1.0005x over previous
import functools
import jax
import jax.numpy as jnp
from jax import lax
from jax.experimental import pallas as pl
from jax.experimental.pallas import tpu as pltpu
from jax.experimental.pallas import tpu_sc as plsc

N = 50000
E = 800000
D = 64
B = 4096
H = 2 * B
NPAD = 50176
BLK = 512
NC = 2
NS = 16
NW = NC * NS
EC = 25088
EPAD = NW * EC
HR = 98
FCAP = EC + 128
CNT_SZ = 8 * NPAD
CSLC = CNT_SZ // NS
ZSLC = CSLC // 4
ACC_R = 8448
ZR = ACC_R // NS

def _vgather(x, idx):
    return lax.gather(
        x, idx[:, None],
        lax.GatherDimensionNumbers(
            offset_dims=(), collapsed_slice_dims=(0,), start_index_map=(0,)),
        slice_sizes=(1,),
        mode=lax.GatherScatterMode.PROMISE_IN_BOUNDS)


_MESH = plsc.VectorSubcoreMesh(
    core_axis_name="c", subcore_axis_name="s", num_cores=NC, num_subcores=NS)


def _stage_a_body(srcp, dstp, xflat, xp2, cnt_out, fpk_out, nb_out,
                  sdh, ddh, cls, tgt, pk, zf, xbuf, nbst, cnt_acc, fp_sh, sem):
    c = lax.axis_index("c")
    s = lax.axis_index("s")
    w = c * NS + s
    fbase = s * FCAP

    def _zf_zero(i, _):
        zf[pl.ds(i * 16, 16)] = jnp.zeros((16,), jnp.float32)
        return 0
    lax.fori_loop(0, ZSLC // 16, _zf_zero, 0)
    for q in range(4):
        pltpu.sync_copy(zf, cnt_acc.at[pl.ds(s * CSLC + q * ZSLC, ZSLC)])

    def _ones(i, _):
        zf[pl.ds(i * 16, 16)] = jnp.ones((16,), jnp.float32)
        return 0
    lax.fori_loop(0, 8, _ones, 0)
    plsc.subcore_barrier()

    def _half(h, cur):
        pltpu.sync_copy(srcp.at[w, h], sdh)
        pltpu.sync_copy(dstp.at[w, h], ddh)

        def _gs(r, _):
            pltpu.make_async_copy(xflat.at[sdh.at[r]], cls.at[r], sem).start()
            return 0
        lax.fori_loop(0, HR, _gs, 0)

        def _gw(r, _):
            pltpu.make_async_copy(xflat.at[sdh.at[0]], cls.at[0], sem).wait()
            return 0
        lax.fori_loop(0, HR, _gw, 0)

        def _flt(i, cur):
            r = i >> 3
            col = (i & 7) * 16
            s16 = sdh[r, pl.ds(col, 16)]
            d16 = ddh[r, pl.ds(col, 16)]
            m = d16 < H
            packed = s16 * (2 * H) + jnp.where(m, d16, 0)
            lane = lax.iota(jnp.int32, 16)
            pos = jnp.where(m, 1, 0).astype(jnp.int32)
            for k in (1, 2, 4, 8):
                sh = _vgather(pos, jnp.maximum(lane - k, 0))
                pos = pos + jnp.where(lane >= k, sh, 0)
            tgt[r, pl.ds(col, 16)] = fbase + jnp.where(
                m, cur + pos - 1, FCAP - 1)
            pk[r, pl.ds(col, 16)] = packed
            pcv = _vgather(pos, jnp.full((16,), 15, jnp.int32))
            return cur + pcv
        cur = lax.fori_loop(0, HR * 8, _flt, cur)

        def _fsc(r, _):
            pltpu.make_async_copy(
                pk.at[r], fp_sh.at[tgt.at[r]], sem).start()
            return 0
        lax.fori_loop(0, HR, _fsc, 0)

        def _fsw(r, _):
            pltpu.make_async_copy(pk.at[0], fp_sh.at[tgt.at[0]], sem).wait()
            return 0
        lax.fori_loop(0, HR, _fsw, 0)

        def _hix(i, _):
            r = i >> 3
            col = (i & 7) * 16
            d16 = ddh[r, pl.ds(col, 16)]
            c16 = cls[r, pl.ds(col, 16)]
            tgt[r, pl.ds(col, 16)] = d16 * 8 + c16
            return 0
        lax.fori_loop(0, HR * 8, _hix, 0)

        def _ss(r, _):
            pltpu.make_async_copy(
                zf.at[pl.ds(0, 128)], cnt_acc.at[tgt.at[r]], sem
            ).start(add=True)
            return 0
        lax.fori_loop(0, HR, _ss, 0)

        def _sw(r, _):
            pltpu.make_async_copy(
                zf.at[pl.ds(0, 128)], cnt_acc.at[tgt.at[0]], sem).wait()
            return 0
        lax.fori_loop(0, HR, _sw, 0)
        return cur

    lane16 = lax.iota(jnp.int32, 16)
    cur = _half(0, jnp.zeros((16,), jnp.int32))
    cur = _half(1, cur)

    nr = jnp.where(w < 8, 13, 12)
    base = w * 12 + jnp.minimum(w, 8)
    pltpu.sync_copy(xp2.at[pl.ds(base * 128, 13 * 128)], xbuf)

    def _nhx(i, _):
        r = i >> 3
        col = (i & 7) * 16
        node = base * 128 + i * 16 + lane16
        xv = xbuf[pl.ds(i * 16, 16)]
        tgt[r, pl.ds(col, 16)] = node * 8 + 4 + xv
        return 0
    lax.fori_loop(0, nr * 8, _nhx, 0)

    def _nsc(r, _):
        pltpu.make_async_copy(
            zf.at[pl.ds(0, 128)], cnt_acc.at[tgt.at[r]], sem).start(add=True)
        return 0
    lax.fori_loop(0, nr, _nsc, 0)

    def _nsw(r, _):
        pltpu.make_async_copy(
            zf.at[pl.ds(0, 128)], cnt_acc.at[tgt.at[0]], sem).wait()
        return 0
    lax.fori_loop(0, nr, _nsw, 0)

    for k in range(8):
        tgt[0, pl.ds(k * 16, 16)] = fbase + cur + lane16 + k * 16
        pk[0, pl.ds(k * 16, 16)] = jnp.full((16,), H, jnp.int32)
    pltpu.sync_copy(pk.at[0], fp_sh.at[tgt.at[0]])

    nb128 = lax.shift_right_logical(cur + 127, 7)

    for k in range(8):
        nbst[pl.ds(k * 16, 16)] = nb128
    pltpu.sync_copy(nbst, nb_out.at[w])
    pltpu.sync_copy(fp_sh.at[pl.ds(fbase, FCAP)], fpk_out.at[w])

    plsc.subcore_barrier()
    pltpu.sync_copy(cnt_acc.at[pl.ds(s * CSLC, CSLC)],
                    cnt_out.at[c, pl.ds(s * CSLC, CSLC)])


_stage_a = functools.partial(
    pl.kernel,
    out_type=(
        jax.ShapeDtypeStruct((NC, CNT_SZ), jnp.float32),
        jax.ShapeDtypeStruct((NW, FCAP), jnp.int32),
        jax.ShapeDtypeStruct((NW, 128), jnp.int32),
    ),
    mesh=_MESH,
    scratch_types=(
        pltpu.VMEM((HR, 128), jnp.int32),
        pltpu.VMEM((HR, 128), jnp.int32),
        pltpu.VMEM((HR, 128), jnp.int32),
        pltpu.VMEM((HR, 128), jnp.int32),
        pltpu.VMEM((HR, 128), jnp.int32),
        pltpu.VMEM((ZSLC,), jnp.float32),
        pltpu.VMEM((13 * 128,), jnp.int32),
        pltpu.VMEM((128,), jnp.int32),
        pltpu.VMEM_SHARED((CNT_SZ,), jnp.float32),
        pltpu.VMEM_SHARED((NS * FCAP,), jnp.int32),
        pltpu.SemaphoreType.DMA,
    ),
)(_stage_a_body)


def _stage_c_body(z1, fpk, nb, s2_out, fp_v, fsr0, fdr0, fsr1, fdr1,
                  rows0, rows1, nbv, acc, sem, addsem0, addsem1):
    c = lax.axis_index("c")
    s = lax.axis_index("s")
    w = c * NS + s

    def _rz_all(i, _):
        r = i >> 3
        col = (i & 7) * 16
        rows0[r, pl.ds(col, 16)] = jnp.zeros((16,), jnp.float32)
        return 0
    lax.fori_loop(0, 128 * 8, _rz_all, 0)
    base = s * ZR
    for k in range(4):
        pltpu.sync_copy(rows0, acc.at[pl.ds(base + k * 128, 128)])
    pltpu.sync_copy(rows0.at[pl.ds(0, 16)], acc.at[pl.ds(base + 512, 16)])
    plsc.subcore_barrier()

    pltpu.sync_copy(fpk.at[w], fp_v)
    pltpu.sync_copy(nb.at[w, pl.ds(0, 16)], nbv)
    n128 = nbv[...][0]

    bufs = ((fsr0, fdr0, rows0, addsem0), (fsr1, fdr1, rows1, addsem1))

    def _batch2(gg, _):
        for p in range(2):
            b = gg * 2 + p
            fsr, fdr, rows, addsem = bufs[p]

            @pl.when(b < n128)
            def _():
                for k in range(8):
                    v = fp_v[pl.ds(b * 128 + k * 16, 16)]
                    fsr[pl.ds(k * 16, 16)] = lax.shift_right_logical(v, 14)
                    fdr[pl.ds(k * 16, 16)] = v & (2 * H - 1)

                @pl.when(b >= 2)
                def _():
                    pltpu.make_async_copy(rows, acc.at[fdr], addsem).wait()

                g = pltpu.make_async_copy(z1.at[fsr], rows, sem)
                g.start()
                g.wait()
                pltpu.make_async_copy(rows, acc.at[fdr], addsem).start(
                    add=True)
        return 0
    lax.fori_loop(0, (n128 + 1) // 2, _batch2, 0)

    odd = (n128 & 1) == 1

    @pl.when((n128 >= 1) & odd)
    def _():
        pltpu.make_async_copy(rows0, acc.at[fdr0], addsem0).wait()

    @pl.when((n128 >= 1) & jnp.logical_not(odd))
    def _():
        pltpu.make_async_copy(rows1, acc.at[fdr1], addsem1).wait()

    @pl.when((n128 >= 2) & odd)
    def _():
        pltpu.make_async_copy(rows1, acc.at[fdr1], addsem1).wait()

    @pl.when((n128 >= 2) & jnp.logical_not(odd))
    def _():
        pltpu.make_async_copy(rows0, acc.at[fdr0], addsem0).wait()

    plsc.subcore_barrier()
    pltpu.sync_copy(acc.at[pl.ds(s * 512, 512)],
                    s2_out.at[c, pl.ds(s * 512, 512)])


_stage_c = functools.partial(
    pl.kernel,
    out_type=jax.ShapeDtypeStruct((NC, H, 128), jnp.float32),
    mesh=_MESH,
    scratch_types=(
        pltpu.VMEM((FCAP,), jnp.int32),
        pltpu.VMEM((128,), jnp.int32),
        pltpu.VMEM((128,), jnp.int32),
        pltpu.VMEM((128,), jnp.int32),
        pltpu.VMEM((128,), jnp.int32),
        pltpu.VMEM((128, 128), jnp.float32),
        pltpu.VMEM((128, 128), jnp.float32),
        pltpu.VMEM((16,), jnp.int32),
        pltpu.VMEM_SHARED((ACC_R, 128), jnp.float32),
        pltpu.SemaphoreType.DMA,
        pltpu.SemaphoreType.DMA,
        pltpu.SemaphoreType.DMA,
    ),
)(_stage_c_body)


def _stage_b_body(cnt_ref, mr_ref, bl1_ref, out_ref):
    cnt = cnt_ref[0] + cnt_ref[1]
    deg = jnp.sum(cnt[:, :4], axis=1, keepdims=True)
    invd = 1.0 / jnp.maximum(deg, 1.0)
    col = lax.broadcasted_iota(jnp.int32, (BLK, 8), 1)
    a = cnt * jnp.where(col < 4, invd, 1.0)
    out = lax.dot_general(a, mr_ref[...], (((1,), (0,)), ((), ())),
                          preferred_element_type=jnp.float32) + bl1_ref[...]
    norm = jnp.sqrt(jnp.sum(out * out, axis=1, keepdims=True))
    out = out / jnp.maximum(norm, 1e-12)
    out_ref[...] = jnp.concatenate(
        [jnp.maximum(out, 0.0), jnp.zeros((BLK, 128 - D), jnp.float32)],
        axis=1)


def _stage_b(cnt2, m1r1, bl1):
    grid = (NPAD // BLK,)
    return pl.pallas_call(
        _stage_b_body,
        grid=grid,
        in_specs=[
            pl.BlockSpec((2, BLK, 8), lambda i: (0, i, 0)),
            pl.BlockSpec((8, D), lambda i: (0, 0)),
            pl.BlockSpec((1, D), lambda i: (0, 0)),
        ],
        out_specs=pl.BlockSpec((BLK, 128), lambda i: (i, 0)),
        out_shape=jax.ShapeDtypeStruct((NPAD, 128), jnp.float32),
    )(cnt2, m1r1, bl1)


def _stage_d_body(s2_ref, cnt_ref, z1_ref, wl2_ref, wr2_ref, bl2_ref,
                  w1_ref, b1_ref, w2_ref, b2_ref, out_ref):
    s = s2_ref[0, :, :D] + s2_ref[1, :, :D]
    cnt = cnt_ref[0] + cnt_ref[1]
    deg = jnp.sum(cnt[:, :4], axis=1, keepdims=True)
    aggr = s * (1.0 / jnp.maximum(deg, 1.0))
    z1s = z1_ref[:, :D]
    dn = (((1,), (1,)), ((), ()))
    out = (lax.dot_general(aggr, wl2_ref[...], dn,
                           preferred_element_type=jnp.float32)
           + lax.dot_general(z1s, wr2_ref[...], dn,
                             preferred_element_type=jnp.float32)
           + bl2_ref[...])
    norm = jnp.sqrt(jnp.sum(out * out, axis=1, keepdims=True))
    z2 = jnp.maximum(out / jnp.maximum(norm, 1e-12), 0.0)
    za = z2[:B]
    zb = z2[B:]
    w1a = w1_ref[:, :D]
    w1b = w1_ref[:, D:]
    h = (lax.dot_general(za, w1a, dn, preferred_element_type=jnp.float32)
         + lax.dot_general(zb, w1b, dn, preferred_element_type=jnp.float32)
         + b1_ref[...])
    h = jnp.maximum(h, 0.0)
    pred = jnp.sum(h * w2_ref[...], axis=1, keepdims=True) + b2_ref[...]
    out_ref[...] = pred


def _stage_d(s2, cnt2h, z1h, wl2, wr2, bl2, w1, b1, w2, b2):
    return pl.pallas_call(
        _stage_d_body,
        grid=(1,),
        in_specs=[
            pl.BlockSpec((2, H, 128), lambda i: (0, 0, 0)),
            pl.BlockSpec((2, H, 8), lambda i: (0, 0, 0)),
            pl.BlockSpec((H, 128), lambda i: (0, 0)),
            pl.BlockSpec((D, D), lambda i: (0, 0)),
            pl.BlockSpec((D, D), lambda i: (0, 0)),
            pl.BlockSpec((1, D), lambda i: (0, 0)),
            pl.BlockSpec((D, 2 * D), lambda i: (0, 0)),
            pl.BlockSpec((1, D), lambda i: (0, 0)),
            pl.BlockSpec((1, D), lambda i: (0, 0)),
            pl.BlockSpec((1, 1), lambda i: (0, 0)),
        ],
        out_specs=pl.BlockSpec((B, 1), lambda i: (0, 0)),
        out_shape=jax.ShapeDtypeStruct((B, 1), jnp.float32),
    )(s2, cnt2h, z1h, wl2, wr2, bl2, w1, b1, w2, b2)


def kernel(x, edge_index, edge_label, node_emb,
           Wl1, bl1, Wr1, Wl2, bl2, Wr2, W1, b1, W2, b2):
    m1r1 = jnp.concatenate([node_emb @ Wl1.T, node_emb @ Wr1.T])
    xflat = x[:, 0]
    xp2 = jnp.pad(xflat, (0, NPAD + 128 - N))

    padn = EPAD - E
    src_p = jnp.concatenate(
        [edge_index[0], jnp.zeros((padn,), jnp.int32)]).reshape(NW, 2, HR, 128)
    dst_p = jnp.concatenate(
        [edge_index[1],
         jnp.full((padn,), NPAD - 1, jnp.int32)]).reshape(NW, 2, HR, 128)

    cnt_flat, fpk, nb = _stage_a(src_p, dst_p, xflat, xp2)
    cnt2 = cnt_flat.reshape(NC, NPAD, 8)

    z1 = _stage_b(cnt2, m1r1, bl1.reshape(1, D))
    s2 = _stage_c(z1, fpk, nb)
    pred = _stage_d(s2, cnt2, z1,
                    Wl2, Wr2, bl2.reshape(1, D),
                    W1, b1.reshape(1, D), W2.reshape(1, D),
                    b2.reshape(1, 1))
    return (pred, edge_label)
